# baseline (speedup 1.0000x reference)
import jax
import jax.numpy as jnp
from jax import lax
from jax.experimental import pallas as pl
from jax.experimental.pallas import tpu as pltpu

B, A, D, M = 128, 64, 8, 96
ATOM_F, BOND_F, FP = 39, 10, 128
NEG = -9e8
MB = 8
R = MB * A
E = MB * A * D
GRID = B // MB

f32 = jnp.float32
HIGH = lax.Precision.HIGHEST


def _dot(x, w):
    return jnp.dot(x, w, preferred_element_type=f32)


def _gdot(oh, v):
    return jnp.dot(oh, v, preferred_element_type=f32, precision=HIGH)


def _gru(x, h, WihT, WhhT, bih, bhh):
    gi = _dot(x, WihT) + bih
    gh = _dot(h, WhhT) + bhh
    r = jax.nn.sigmoid(gi[:, :FP] + gh[:, :FP])
    z = jax.nn.sigmoid(gi[:, FP:2 * FP] + gh[:, FP:2 * FP])
    n = jnp.tanh(gi[:, 2 * FP:] + r * gh[:, 2 * FP:])
    return (1.0 - z) * n + z * h


def _elu(x):
    xn = jnp.minimum(x, 0.0)
    em1 = jnp.tanh(0.5 * xn) * (jnp.exp(xn) + 1.0)
    return jnp.where(x > 0, x, em1)


def _softmax_mid(score):
    m = jnp.max(score, axis=1, keepdims=True)
    e = jnp.exp(score - m)
    return e / jnp.sum(e, axis=1, keepdims=True)


def _fused_kernel(af_ref, bf_ref, an_ref, bn_ref, mask_ref,
                  W_atom_ref, b_atom_ref, W_nei_ref, bnei_ref,
                  alW_ref, alb_ref, atW_ref, atb_ref,
                  gih_ref, ghh_ref, gbih_ref, gbhh_ref,
                  malW_ref, malb_ref, matW_ref, matb_ref,
                  mgih_ref, mghh_ref, mgbih_ref, mgbhh_ref,
                  dW1_ref, db1_ref, dW2_ref, db2_ref, dW3_ref, db3_ref,
                  out_ref):
    af = af_ref[...].reshape(R, ATOM_F)
    bf = bf_ref[...].reshape(MB * M, BOND_F)
    an = an_ref[...]
    bn = bn_ref[...]

    afp = jax.nn.leaky_relu(_dot(af, W_atom_ref[...]) + b_atom_ref[...])

    iota_a = lax.broadcasted_iota(jnp.int32, (A, D, A), 2)
    iota_b = lax.broadcasted_iota(jnp.int32, (A, D, M), 2)
    oh_a = [(an[m][:, :, None] == iota_a).astype(f32).reshape(A * D, A)
            for m in range(MB)]
    oh_b = [(bn[m][:, :, None] == iota_b).astype(f32).reshape(A * D, M)
            for m in range(MB)]

    negm = jnp.where(an == A - 1, NEG, 0.0).astype(f32).reshape(R, D)[:, :, None]
    attm = (an != A - 1).astype(f32).reshape(R, D)[:, :, None]

    feat = jnp.concatenate(
        [jnp.concatenate(
            [_gdot(oh_a[m], af[m * A:(m + 1) * A]),
             _gdot(oh_b[m], bf[m * M:(m + 1) * M])], axis=1)
         for m in range(MB)], axis=0)
    nfp = jax.nn.leaky_relu(_dot(feat, W_nei_ref[...]) + bnei_ref[...])

    for i in range(3):
        if i > 0:
            nfp = jnp.concatenate(
                [_gdot(oh_a[m], afp[m * A:(m + 1) * A]) for m in range(MB)],
                axis=0)
        atom_exp = jnp.broadcast_to(
            afp.reshape(R, 1, FP), (R, D, FP)).reshape(E, FP)
        falign = jnp.concatenate([atom_exp, nfp], axis=1)
        score = (jax.nn.leaky_relu(_dot(falign, alW_ref[i]) + alb_ref[i])
                 .reshape(R, D, 1) + negm)
        attw = _softmax_mid(score) * attm
        nt = _dot(nfp, atW_ref[i]) + atb_ref[i]
        ctx = _elu(jnp.sum(attw * nt.reshape(R, D, FP), axis=1))
        afp = _gru(ctx, afp, gih_ref[i], ghh_ref[i], gbih_ref[i], gbhh_ref[i])

    mask2 = mask_ref[...]
    afp3 = afp.reshape(MB, A, FP)
    super_ = jnp.sum(afp3 * mask2[:, :, None], axis=1)
    molneg = jnp.where(mask2 == 0.0, NEG, 0.0).astype(f32)[:, :, None]
    molmask = mask2[:, :, None]
    act = super_
    for _ in range(2):
        super_exp = jnp.broadcast_to(
            super_.reshape(MB, 1, FP), (MB, A, FP)).reshape(R, FP)
        malign = jnp.concatenate([super_exp, afp], axis=1)
        score = (jax.nn.leaky_relu(_dot(malign, malW_ref[...]) + malb_ref[...])
                 .reshape(MB, A, 1) + molneg)
        attw = _softmax_mid(score) * molmask
        atom_t = _dot(afp, matW_ref[...]) + matb_ref[...]
        ctx = _elu(jnp.sum(attw * atom_t.reshape(MB, A, FP), axis=1))
        super_ = _gru(ctx, super_, mgih_ref[...], mghh_ref[...],
                      mgbih_ref[...], mgbhh_ref[...])
        act = jax.nn.relu(super_)

    h1 = jax.nn.relu(_dot(act, dW1_ref[...]) + db1_ref[...])
    h2 = jax.nn.relu(_dot(h1, dW2_ref[...]) + db2_ref[...])
    out_ref[...] = _dot(h2, dW3_ref[...]) + db3_ref[...]


def kernel(atom_features, bond_features, atom_neighbor_list, bond_neighbor_list,
           atom_mask, W_atom, b_atom, W_nei, b_nei, align_W, align_b,
           attend_W, attend_b, gru_Wih, gru_Whh, gru_bih, gru_bhh,
           mol_align_W, mol_align_b, mol_attend_W, mol_attend_b,
           mol_gru_Wih, mol_gru_Whh, mol_gru_bih, mol_gru_bhh,
           dnn_W1, dnn_b1, dnn_W2, dnn_b2, dnn_W3, dnn_b3):
    alb = align_b.reshape(3, 1, 1)
    atb = attend_b.reshape(3, 1, FP)
    gihT = jnp.swapaxes(gru_Wih, 1, 2)
    ghhT = jnp.swapaxes(gru_Whh, 1, 2)
    gbih = gru_bih.reshape(3, 1, 3 * FP)
    gbhh = gru_bhh.reshape(3, 1, 3 * FP)
    malb = mol_align_b.reshape(1, 1)
    matb = mol_attend_b.reshape(1, FP)
    mgihT = mol_gru_Wih.T
    mghhT = mol_gru_Whh.T
    mgbih = mol_gru_bih.reshape(1, 3 * FP)
    mgbhh = mol_gru_bhh.reshape(1, 3 * FP)
    db1 = dnn_b1.reshape(1, 512)
    db2 = dnn_b2.reshape(1, 128)
    db3 = dnn_b3.reshape(1, 1)
    b_atom2 = b_atom.reshape(1, FP)
    b_nei2 = b_nei.reshape(1, FP)

    an32 = atom_neighbor_list.astype(jnp.int32)
    bn32 = bond_neighbor_list.astype(jnp.int32)

    def blk(shape, imap):
        return pl.BlockSpec(shape, imap)

    full = lambda arr: pl.BlockSpec(arr.shape, lambda i: (0,) * arr.ndim)

    in_specs = [
        blk((MB, A, ATOM_F), lambda i: (i, 0, 0)),
        blk((MB, M, BOND_F), lambda i: (i, 0, 0)),
        blk((MB, A, D), lambda i: (i, 0, 0)),
        blk((MB, A, D), lambda i: (i, 0, 0)),
        blk((MB, A), lambda i: (i, 0)),
    ]
    weights = [W_atom, b_atom2, W_nei, b_nei2,
               align_W, alb, attend_W, atb,
               gihT, ghhT, gbih, gbhh,
               mol_align_W, malb, mol_attend_W, matb,
               mgihT, mghhT, mgbih, mgbhh,
               dnn_W1, db1, dnn_W2, db2, dnn_W3, db3]
    in_specs += [full(w) for w in weights]

    out = pl.pallas_call(
        _fused_kernel,
        grid=(GRID,),
        in_specs=in_specs,
        out_specs=pl.BlockSpec((MB, 1), lambda i: (i, 0)),
        out_shape=jax.ShapeDtypeStruct((B, 1), f32),
        compiler_params=pltpu.CompilerParams(
            dimension_semantics=("arbitrary",)),
    )(atom_features, bond_features, an32, bn32, atom_mask, *weights)
    return out
